# Initial kernel scaffold; baseline (speedup 1.0000x reference)
import jax
import jax.numpy as jnp
from jax.experimental import pallas as pl


def kernel(x, edge_index, batch, W1, b1, W2, b2):
    raise NotImplementedError("write your pallas kernel here")


# speedup vs baseline: 11.0100x; 11.0100x over previous
import functools

import jax
import jax.numpy as jnp
from jax import lax
from jax.experimental import pallas as pl
from jax.experimental.pallas import tpu as pltpu
from jax.experimental.pallas import tpu_sc as plsc

NC = 2
NS = 16
NW = NC * NS
CHUNK = 128


def _mesh():
  return plsc.VectorSubcoreMesh(core_axis_name="c", subcore_axis_name="s")


def _make_deg_kernel(nchunk, nrows, rpt):
  @functools.partial(
      pl.kernel,
      out_type=jax.ShapeDtypeStruct((NC, nrows), jnp.float32),
      mesh=_mesh(),
      compiler_params=pltpu.CompilerParams(use_tc_tiling_on_sc=False),
      scratch_types=[
          pltpu.VMEM((nchunk, CHUNK), jnp.int32),
          pltpu.VMEM((CHUNK,), jnp.float32),
          pltpu.VMEM_SHARED((nrows,), jnp.float32),
      ],
  )
  def deg_kernel(dst_hbm, zrow_hbm, ones_hbm, out_hbm, idx_v, ones_v, deg_sh):
    c = lax.axis_index("c")
    s = lax.axis_index("s")
    w = c * NS + s
    pltpu.sync_copy(dst_hbm.at[pl.ds(w * nchunk, nchunk)], idx_v)
    pltpu.sync_copy(ones_hbm, ones_v)
    pltpu.sync_copy(zrow_hbm, deg_sh.at[pl.ds(s * rpt, rpt)])
    plsc.subcore_barrier()

    def chunk_body(j, carry):
      pltpu.sync_copy(ones_v, deg_sh.at[idx_v.at[j]], add=True)
      return carry

    lax.fori_loop(0, nchunk, chunk_body, 0)
    plsc.subcore_barrier()
    pltpu.sync_copy(deg_sh.at[pl.ds(s * rpt, rpt)],
                    out_hbm.at[c, pl.ds(s * rpt, rpt)])

  return deg_kernel


def _make_agg_kernel(nchunk, nrows, rpt, dim_h):
  @functools.partial(
      pl.kernel,
      out_type=jax.ShapeDtypeStruct((NC, nrows, dim_h), jnp.float32),
      mesh=_mesh(),
      scratch_types=[
          pltpu.VMEM((nchunk, CHUNK), jnp.int32),
          pltpu.VMEM((nchunk, CHUNK), jnp.int32),
          pltpu.VMEM((CHUNK, dim_h), jnp.float32),
          pltpu.VMEM_SHARED((nrows, dim_h), jnp.float32),
          pltpu.SemaphoreType.DMA,
      ],
  )
  def agg_kernel(hp_hbm, src_hbm, dst_hbm, zrows_hbm, out_hbm,
                 sidx_v, didx_v, buf_v, agg_sh, sem):
    c = lax.axis_index("c")
    s = lax.axis_index("s")
    w = c * NS + s
    pltpu.sync_copy(src_hbm.at[pl.ds(w * nchunk, nchunk)], sidx_v)
    pltpu.sync_copy(dst_hbm.at[pl.ds(w * nchunk, nchunk)], didx_v)
    pltpu.sync_copy(zrows_hbm, agg_sh.at[pl.ds(s * rpt, rpt)])
    plsc.subcore_barrier()

    def chunk_body(j, carry):
      pltpu.async_copy(hp_hbm.at[sidx_v.at[j]], buf_v, sem).wait()
      pltpu.sync_copy(buf_v, agg_sh.at[didx_v.at[j]], add=True)
      return carry

    lax.fori_loop(0, nchunk, chunk_body, 0)
    plsc.subcore_barrier()
    pltpu.sync_copy(agg_sh.at[pl.ds(s * rpt, rpt)],
                    out_hbm.at[c, pl.ds(s * rpt, rpt)])

  return agg_kernel


def _hp_body(x_ref, w1_ref, dega_ref, degb_ref, o_ref):
  deg = dega_ref[...] + degb_ref[...] + 1.0
  d = lax.rsqrt(deg)
  o_ref[...] = jnp.dot(x_ref[...], w1_ref[...],
                       preferred_element_type=jnp.float32) * d


def _head_body(sa_ref, sb_ref, hp_ref, dega_ref, degb_ref, bf_ref, b1_ref,
               w2_ref, b2_ref, o_ref, sums, counts):
  j = pl.program_id(0)

  @pl.when(j == 0)
  def _init():
    sums[...] = jnp.zeros_like(sums)
    counts[...] = jnp.zeros_like(counts)

  d = lax.rsqrt(dega_ref[...] + degb_ref[...] + 1.0)
  h2 = d * (sa_ref[...] + sb_ref[...] + hp_ref[...]) + b1_ref[...]
  h2 = jnp.maximum(h2, 0.0)
  gids = lax.broadcasted_iota(jnp.int32, (1, sums.shape[0]), 1
                              ).astype(jnp.float32)
  onehot = (bf_ref[...] == gids).astype(jnp.float32)
  sums[...] += jnp.dot(onehot.T, h2, preferred_element_type=jnp.float32)
  counts[...] += jnp.sum(onehot, axis=0, keepdims=True)

  @pl.when(j == pl.num_programs(0) - 1)
  def _finish():
    hg = sums[...] / jnp.maximum(counts[...], 1.0).T
    logits = jnp.dot(hg, w2_ref[...],
                     preferred_element_type=jnp.float32) + b2_ref[...]
    m = jnp.max(logits, axis=1, keepdims=True)
    lse = jnp.log(jnp.sum(jnp.exp(logits - m), axis=1, keepdims=True)) + m
    o_ref[...] = logits - lse


def kernel(x, edge_index, batch, W1, b1, W2, b2):
  n, d_feat = x.shape
  dim_h = W1.shape[1]
  n_classes = W2.shape[1]
  e = edge_index.shape[1]
  n_graphs = 128

  nchunk = -(-e // (NW * CHUNK))
  nchunk = (nchunk + 7) // 8 * 8
  e_pad = NW * CHUNK * nchunk
  src = edge_index[0].astype(jnp.int32)
  dst = edge_index[1].astype(jnp.int32)
  pad = e_pad - e
  src2d = jnp.concatenate([src, jnp.zeros((pad,), jnp.int32)]
                          ).reshape(NW * nchunk, CHUNK)
  dst2d = jnp.concatenate([dst, jnp.full((pad,), n, jnp.int32)]
                          ).reshape(NW * nchunk, CHUNK)

  rpt = ((-(-(n + 1) // NS)) + 7) // 8 * 8
  nrows = rpt * NS

  zrow = jnp.zeros((rpt,), jnp.float32)
  ones = jnp.ones((CHUNK,), jnp.float32)
  deg_parts = _make_deg_kernel(nchunk, nrows, rpt)(dst2d, zrow, ones)
  dega = deg_parts[0, :n].reshape(n, 1)
  degb = deg_parts[1, :n].reshape(n, 1)

  nb = 10
  blk = n // nb
  hp = pl.pallas_call(
      _hp_body,
      grid=(nb,),
      in_specs=[
          pl.BlockSpec((blk, d_feat), lambda i: (i, 0)),
          pl.BlockSpec((d_feat, dim_h), lambda i: (0, 0)),
          pl.BlockSpec((blk, 1), lambda i: (i, 0)),
          pl.BlockSpec((blk, 1), lambda i: (i, 0)),
      ],
      out_specs=pl.BlockSpec((blk, dim_h), lambda i: (i, 0)),
      out_shape=jax.ShapeDtypeStruct((n, dim_h), jnp.float32),
  )(x, W1, dega, degb)

  zrows = jnp.zeros((rpt, dim_h), jnp.float32)
  agg_parts = _make_agg_kernel(nchunk, nrows, rpt, dim_h)(
      hp, src2d, dst2d, zrows)
  sa = agg_parts[0, :n, :]
  sb = agg_parts[1, :n, :]

  bf = batch.astype(jnp.float32).reshape(n, 1)
  out = pl.pallas_call(
      _head_body,
      grid=(nb,),
      in_specs=[
          pl.BlockSpec((blk, dim_h), lambda i: (i, 0)),
          pl.BlockSpec((blk, dim_h), lambda i: (i, 0)),
          pl.BlockSpec((blk, dim_h), lambda i: (i, 0)),
          pl.BlockSpec((blk, 1), lambda i: (i, 0)),
          pl.BlockSpec((blk, 1), lambda i: (i, 0)),
          pl.BlockSpec((blk, 1), lambda i: (i, 0)),
          pl.BlockSpec((1, dim_h), lambda i: (0, 0)),
          pl.BlockSpec((dim_h, n_classes), lambda i: (0, 0)),
          pl.BlockSpec((1, n_classes), lambda i: (0, 0)),
      ],
      out_specs=pl.BlockSpec((n_graphs, n_classes), lambda i: (0, 0)),
      out_shape=jax.ShapeDtypeStruct((n_graphs, n_classes), jnp.float32),
      scratch_shapes=[
          pltpu.VMEM((n_graphs, dim_h), jnp.float32),
          pltpu.VMEM((1, n_graphs), jnp.float32),
      ],
  )(sa, sb, hp, dega, degb, bf, b1.reshape(1, dim_h), W2,
    b2.reshape(1, n_classes))
  return out


# speedup vs baseline: 11.7987x; 1.0716x over previous
import functools

import jax
import jax.numpy as jnp
from jax import lax
from jax.experimental import pallas as pl
from jax.experimental.pallas import tpu as pltpu
from jax.experimental.pallas import tpu_sc as plsc

NC = 2
NS = 16
NW = NC * NS
CHUNK = 128


def _mesh():
  return plsc.VectorSubcoreMesh(core_axis_name="c", subcore_axis_name="s")


def _make_deg_kernel(nchunk, nrows, rpt):
  @functools.partial(
      pl.kernel,
      out_type=jax.ShapeDtypeStruct((NC, nrows), jnp.float32),
      mesh=_mesh(),
      compiler_params=pltpu.CompilerParams(use_tc_tiling_on_sc=False),
      scratch_types=[
          pltpu.VMEM((nchunk, CHUNK), jnp.int32),
          pltpu.VMEM((CHUNK,), jnp.float32),
          pltpu.VMEM_SHARED((nrows,), jnp.float32),
      ],
  )
  def deg_kernel(dst_hbm, zrow_hbm, ones_hbm, out_hbm, idx_v, ones_v, deg_sh):
    c = lax.axis_index("c")
    s = lax.axis_index("s")
    w = c * NS + s
    pltpu.sync_copy(dst_hbm.at[pl.ds(w * nchunk, nchunk)], idx_v)
    pltpu.sync_copy(ones_hbm, ones_v)
    pltpu.sync_copy(zrow_hbm, deg_sh.at[pl.ds(s * rpt, rpt)])
    plsc.subcore_barrier()

    def chunk_body(j, carry):
      pltpu.sync_copy(ones_v, deg_sh.at[idx_v.at[j]], add=True)
      return carry

    lax.fori_loop(0, nchunk, chunk_body, 0)
    plsc.subcore_barrier()
    pltpu.sync_copy(deg_sh.at[pl.ds(s * rpt, rpt)],
                    out_hbm.at[c, pl.ds(s * rpt, rpt)])

  return deg_kernel


def _make_agg_kernel(nchunk, nrows, rpt, dim_h):
  nhalf = nchunk // 2

  @functools.partial(
      pl.kernel,
      out_type=jax.ShapeDtypeStruct((NC, nrows, dim_h), jnp.float32),
      mesh=_mesh(),
      scratch_types=[
          pltpu.VMEM((nhalf, CHUNK), jnp.int32),
          pltpu.VMEM((nhalf, CHUNK), jnp.int32),
          pltpu.VMEM((CHUNK, dim_h), jnp.float32),
          pltpu.VMEM((CHUNK, dim_h), jnp.float32),
          pltpu.VMEM_SHARED((nrows, dim_h), jnp.float32),
          pltpu.SemaphoreType.DMA,
          pltpu.SemaphoreType.DMA,
      ],
  )
  def agg_kernel(hp_hbm, src_hbm, dst_hbm, zrows_hbm, out_hbm,
                 sidx_v, didx_v, buf0, buf1, agg_sh, gsem0, gsem1):
    c = lax.axis_index("c")
    s = lax.axis_index("s")
    w = c * NS + s
    pltpu.sync_copy(zrows_hbm, agg_sh.at[pl.ds(s * rpt, rpt)])
    plsc.subcore_barrier()

    bufs = (buf0, buf1)
    gsems = (gsem0, gsem1)

    def wait_gather(b, j):
      pltpu.make_async_copy(hp_hbm.at[sidx_v.at[j]], bufs[b], gsems[b]).wait()

    for h in range(2):
      base = w * nchunk + h * nhalf
      pltpu.sync_copy(src_hbm.at[pl.ds(base, nhalf)], sidx_v)
      pltpu.sync_copy(dst_hbm.at[pl.ds(base, nhalf)], didx_v)
      pltpu.async_copy(hp_hbm.at[sidx_v.at[0]], buf0, gsem0)

      def pair_body(i, carry):
        for b in range(2):
          j = 2 * i + b
          nb = 1 - b
          wait_gather(b, j)

          @pl.when(j + 1 < nhalf)
          def _():
            pltpu.async_copy(hp_hbm.at[sidx_v.at[j + 1]], bufs[nb], gsems[nb])

          pltpu.sync_copy(bufs[b], agg_sh.at[didx_v.at[j]], add=True)
        return carry

      lax.fori_loop(0, nhalf // 2, pair_body, 0)

    plsc.subcore_barrier()
    pltpu.sync_copy(agg_sh.at[pl.ds(s * rpt, rpt)],
                    out_hbm.at[c, pl.ds(s * rpt, rpt)])

  return agg_kernel


def _hp_body(x_ref, w1_ref, dega_ref, degb_ref, o_ref):
  deg = dega_ref[...] + degb_ref[...] + 1.0
  d = lax.rsqrt(deg)
  o_ref[...] = jnp.dot(x_ref[...], w1_ref[...],
                       preferred_element_type=jnp.float32) * d


def _head_body(sa_ref, sb_ref, hp_ref, dega_ref, degb_ref, bf_ref, b1_ref,
               w2_ref, b2_ref, o_ref, sums, counts):
  j = pl.program_id(0)

  @pl.when(j == 0)
  def _init():
    sums[...] = jnp.zeros_like(sums)
    counts[...] = jnp.zeros_like(counts)

  d = lax.rsqrt(dega_ref[...] + degb_ref[...] + 1.0)
  h2 = d * (sa_ref[...] + sb_ref[...] + hp_ref[...]) + b1_ref[...]
  h2 = jnp.maximum(h2, 0.0)
  gids = lax.broadcasted_iota(jnp.int32, (1, sums.shape[0]), 1
                              ).astype(jnp.float32)
  onehot = (bf_ref[...] == gids).astype(jnp.float32)
  sums[...] += jnp.dot(onehot.T, h2, preferred_element_type=jnp.float32)
  counts[...] += jnp.sum(onehot, axis=0, keepdims=True)

  @pl.when(j == pl.num_programs(0) - 1)
  def _finish():
    hg = sums[...] / jnp.maximum(counts[...], 1.0).T
    logits = jnp.dot(hg, w2_ref[...],
                     preferred_element_type=jnp.float32) + b2_ref[...]
    m = jnp.max(logits, axis=1, keepdims=True)
    lse = jnp.log(jnp.sum(jnp.exp(logits - m), axis=1, keepdims=True)) + m
    o_ref[...] = logits - lse


def kernel(x, edge_index, batch, W1, b1, W2, b2):
  n, d_feat = x.shape
  dim_h = W1.shape[1]
  n_classes = W2.shape[1]
  e = edge_index.shape[1]
  n_graphs = 128

  nchunk = -(-e // (NW * CHUNK))
  nchunk = (nchunk + 7) // 8 * 8
  e_pad = NW * CHUNK * nchunk
  src = edge_index[0].astype(jnp.int32)
  dst = edge_index[1].astype(jnp.int32)
  pad = e_pad - e
  src2d = jnp.concatenate([src, jnp.zeros((pad,), jnp.int32)]
                          ).reshape(NW * nchunk, CHUNK)
  dst2d = jnp.concatenate([dst, jnp.full((pad,), n, jnp.int32)]
                          ).reshape(NW * nchunk, CHUNK)

  rpt = ((-(-(n + 1) // NS)) + 7) // 8 * 8
  nrows = rpt * NS

  zrow = jnp.zeros((rpt,), jnp.float32)
  ones = jnp.ones((CHUNK,), jnp.float32)
  deg_parts = _make_deg_kernel(nchunk, nrows, rpt)(dst2d, zrow, ones)
  dega = deg_parts[0, :n].reshape(n, 1)
  degb = deg_parts[1, :n].reshape(n, 1)

  nb = 10
  blk = n // nb
  hp = pl.pallas_call(
      _hp_body,
      grid=(nb,),
      in_specs=[
          pl.BlockSpec((blk, d_feat), lambda i: (i, 0)),
          pl.BlockSpec((d_feat, dim_h), lambda i: (0, 0)),
          pl.BlockSpec((blk, 1), lambda i: (i, 0)),
          pl.BlockSpec((blk, 1), lambda i: (i, 0)),
      ],
      out_specs=pl.BlockSpec((blk, dim_h), lambda i: (i, 0)),
      out_shape=jax.ShapeDtypeStruct((n, dim_h), jnp.float32),
  )(x, W1, dega, degb)

  zrows = jnp.zeros((rpt, dim_h), jnp.float32)
  agg_parts = _make_agg_kernel(nchunk, nrows, rpt, dim_h)(
      hp, src2d, dst2d, zrows)
  sa = agg_parts[0, :n, :]
  sb = agg_parts[1, :n, :]

  bf = batch.astype(jnp.float32).reshape(n, 1)
  out = pl.pallas_call(
      _head_body,
      grid=(nb,),
      in_specs=[
          pl.BlockSpec((blk, dim_h), lambda i: (i, 0)),
          pl.BlockSpec((blk, dim_h), lambda i: (i, 0)),
          pl.BlockSpec((blk, dim_h), lambda i: (i, 0)),
          pl.BlockSpec((blk, 1), lambda i: (i, 0)),
          pl.BlockSpec((blk, 1), lambda i: (i, 0)),
          pl.BlockSpec((blk, 1), lambda i: (i, 0)),
          pl.BlockSpec((1, dim_h), lambda i: (0, 0)),
          pl.BlockSpec((dim_h, n_classes), lambda i: (0, 0)),
          pl.BlockSpec((1, n_classes), lambda i: (0, 0)),
      ],
      out_specs=pl.BlockSpec((n_graphs, n_classes), lambda i: (0, 0)),
      out_shape=jax.ShapeDtypeStruct((n_graphs, n_classes), jnp.float32),
      scratch_shapes=[
          pltpu.VMEM((n_graphs, dim_h), jnp.float32),
          pltpu.VMEM((1, n_graphs), jnp.float32),
      ],
  )(sa, sb, hp, dega, degb, bf, b1.reshape(1, dim_h), W2,
    b2.reshape(1, n_classes))
  return out
